# baseline (speedup 1.0000x reference)
import functools

import jax
import jax.numpy as jnp
from jax import lax
from jax.experimental import pallas as pl
from jax.experimental.pallas import tpu as pltpu
from jax.experimental.pallas import tpu_sc as plsc

_N = 10000
_NP = 10240
_E = 160000
_EP = 163840
_D = 128
_R2 = 401
_RP = 408
_C = 80
_CPT0 = 112
_CPT1 = 16
_RPT = _NP // 16


def _tbl_body(x_ref, rp_ref, w_ref, xt_ref, rt_ref):
    w = w_ref[0]
    xt_ref[0] = jnp.dot(x_ref[...], w, preferred_element_type=jnp.float32)
    rt_ref[0] = -jnp.dot(rp_ref[...], w, preferred_element_type=jnp.float32)


def _tbl_call(x, rp, w3):
    return pl.pallas_call(
        _tbl_body,
        grid=(3,),
        in_specs=[
            pl.BlockSpec((_NP, _D), lambda k: (0, 0)),
            pl.BlockSpec((_RP, _D), lambda k: (0, 0)),
            pl.BlockSpec((1, _D, _D), lambda k: (k, 0, 0)),
        ],
        out_specs=[
            pl.BlockSpec((1, _NP, _D), lambda k: (k, 0, 0)),
            pl.BlockSpec((1, _RP, _D), lambda k: (k, 0, 0)),
        ],
        out_shape=[
            jax.ShapeDtypeStruct((3, _NP, _D), jnp.float32),
            jax.ShapeDtypeStruct((3, _RP, _D), jnp.float32),
        ],
    )(x, rp, w3)


def _bn_body(p_ref, g_ref, b_ref, rp_ref, rw_ref, x_ref, rn_ref):
    h = p_ref[0] + p_ref[1]
    m = jnp.sum(h, axis=0, keepdims=True) * (1.0 / _N)
    d = h - m
    v = jnp.sum(d * d, axis=0, keepdims=True) * (1.0 / _N) \
        - (float(_NP - _N) / _N) * (m * m)
    xn = jnp.tanh(d / jnp.sqrt(v + 1e-5) * g_ref[...] + b_ref[...])
    row = lax.broadcasted_iota(jnp.int32, (_NP, 1), 0)
    x_ref[...] = jnp.where(row < _N, xn, 0.0)
    rn_ref[...] = jnp.dot(rp_ref[...], rw_ref[...],
                          preferred_element_type=jnp.float32)


def _bn_call(p, g, b, rp, rw):
    return pl.pallas_call(
        _bn_body,
        out_shape=[
            jax.ShapeDtypeStruct((_NP, _D), jnp.float32),
            jax.ShapeDtypeStruct((_RP, _D), jnp.float32),
        ],
    )(p, g, b, rp, rw)


@functools.cache
def _make_sc_scatter():
    mesh = plsc.VectorSubcoreMesh(core_axis_name="c", subcore_axis_name="s")

    @functools.partial(
        pl.kernel,
        mesh=mesh,
        out_type=jax.ShapeDtypeStruct((2 * _NP, _D), jnp.float32),
        scratch_types=[
            pltpu.VMEM((_C,), jnp.int32),
            pltpu.VMEM((_C,), jnp.int32),
            pltpu.VMEM((_C,), jnp.int32),
            pltpu.VMEM((_C,), jnp.int32),
            pltpu.VMEM((_C,), jnp.int32),
            pltpu.VMEM((_C,), jnp.int32),
            pltpu.VMEM((_C, _D), jnp.float32),
            pltpu.VMEM((_C, _D), jnp.float32),
            pltpu.VMEM((_C, _D), jnp.float32),
            pltpu.VMEM((_C, _D), jnp.float32),
            pltpu.VMEM_SHARED((_NP, _D), jnp.float32),
            pltpu.SemaphoreType.DMA,
            pltpu.SemaphoreType.DMA,
            pltpu.SemaphoreType.DMA,
            pltpu.SemaphoreType.DMA,
        ],
    )
    def _sc_scatter(xtbl, nrtbl, gsrc, grel, dste, zer, out,
                    gi0, ri0, di0, gi1, ri1, di1, ra0, rb0, ra1, rb1,
                    acc, si0, si1, sg0, sg1):
        cid = lax.axis_index("c")
        sid = lax.axis_index("s")
        pltpu.sync_copy(zer, acc.at[pl.ds(sid * _RPT, _RPT)])
        ncht = jnp.where(cid == 0, _CPT0, _CPT1)
        cbase = jnp.where(cid == 0, sid * _CPT0, 16 * _CPT0 + sid * _CPT1)
        base = cbase * _C
        isets = ((gi0, ri0, di0, si0), (gi1, ri1, di1, si1))
        rsets = ((ra0, rb0, sg0), (ra1, rb1, sg1))

        def fire_idx(j, s):
            gi, ri, di, si = isets[s]
            o = pl.multiple_of(base + j * _C, 8)
            pltpu.async_copy(gsrc.at[pl.ds(o, _C)], gi, si)
            pltpu.async_copy(grel.at[pl.ds(o, _C)], ri, si)
            pltpu.async_copy(dste.at[pl.ds(o, _C)], di, si)

        def drain_idx(s):
            gi, ri, di, si = isets[s]
            pltpu.make_async_copy(gsrc.at[pl.ds(0, _C)], gi, si).wait()
            pltpu.make_async_copy(grel.at[pl.ds(0, _C)], ri, si).wait()
            pltpu.make_async_copy(dste.at[pl.ds(0, _C)], di, si).wait()

        def fire_rows(s):
            gi, ri, _, _ = isets[s]
            ra, rb, sg = rsets[s]
            pltpu.async_copy(xtbl.at[gi], ra, sg)
            pltpu.async_copy(nrtbl.at[ri], rb, sg)

        def drain_rows(s):
            gi, ri, _, _ = isets[s]
            ra, rb, sg = rsets[s]
            pltpu.make_async_copy(xtbl.at[gi], ra, sg).wait()
            pltpu.make_async_copy(nrtbl.at[ri], rb, sg).wait()

        fire_idx(0, 0)
        fire_idx(1, 1)
        drain_idx(0)
        fire_rows(0)
        plsc.subcore_barrier()

        def body(j2, carry):
            for b in range(2):
                j = j2 * 2 + b
                nb = 1 - b

                @pl.when(j + 1 < ncht)
                def _():
                    drain_idx(nb)
                drain_rows(b)

                @pl.when(j + 1 < ncht)
                def _():
                    fire_rows(nb)
                di = isets[b][2]
                ra, rb, _ = rsets[b]
                pltpu.sync_copy(ra, acc.at[di], add=True)
                pltpu.sync_copy(rb, acc.at[di], add=True)

                @pl.when(j + 2 < ncht)
                def _():
                    fire_idx(j + 2, b)
            return carry

        lax.fori_loop(0, ncht // 2, body, 0)
        plsc.subcore_barrier()
        off = pl.multiple_of(cid * _NP + sid * _RPT, _RPT)
        pltpu.sync_copy(acc.at[pl.ds(sid * _RPT, _RPT)],
                        out.at[pl.ds(off, _RPT)])

    return _sc_scatter


def kernel(ent_ids, edge_index, edge_type, y, entity_embeds, rel_embeds,
           self_rel_embed, weights1, relation_weight1, bn1_gamma, bn1_beta,
           weights2, relation_weight2, bn2_gamma, bn2_beta):
    f32 = jnp.float32
    x0 = jnp.take(entity_embeds, ent_ids, axis=0)
    x0 = jnp.concatenate([x0, jnp.zeros((_NP - _N, _D), f32)], axis=0)
    r0 = jnp.concatenate(
        [rel_embeds, -rel_embeds, self_rel_embed,
         jnp.zeros((_RP - _R2, _D), f32)], axis=0)
    src = edge_index[0]
    dstv = edge_index[1]
    pad = _EP - _E
    gsrc = jnp.concatenate([y * _NP + src, jnp.full((pad,), _N, jnp.int32)])
    grel = jnp.concatenate([y * _RP + edge_type,
                            jnp.full((pad,), _R2, jnp.int32)])
    dste = jnp.concatenate([dstv, jnp.full((pad,), _N + 100, jnp.int32)])
    zer = jnp.zeros((_RPT, _D), f32)
    g1 = bn1_gamma.reshape(1, _D)
    b1 = bn1_beta.reshape(1, _D)
    g2 = bn2_gamma.reshape(1, _D)
    b2 = bn2_beta.reshape(1, _D)

    sc_scatter = _make_sc_scatter()
    xt, rt = _tbl_call(x0, r0, weights1)
    part = sc_scatter(xt.reshape(3 * _NP, _D), rt.reshape(3 * _RP, _D),
                      gsrc, grel, dste, zer)
    x1, r1 = _bn_call(part.reshape(2, _NP, _D), g1, b1, r0, relation_weight1)

    xt2, rt2 = _tbl_call(x1, r1, weights2)
    part2 = sc_scatter(xt2.reshape(3 * _NP, _D), rt2.reshape(3 * _RP, _D),
                       gsrc, grel, dste, zer)
    x2, r2 = _bn_call(part2.reshape(2, _NP, _D), g2, b2, r1, relation_weight2)

    return x2[:_N], r2[:_R2]


# speedup vs baseline: 1.0231x; 1.0231x over previous
import functools

import jax
import jax.numpy as jnp
from jax import lax
from jax.experimental import pallas as pl
from jax.experimental.pallas import tpu as pltpu
from jax.experimental.pallas import tpu_sc as plsc

_N = 10000
_NP = 10240
_E = 160000
_EP = 163840
_D = 128
_R2 = 401
_RP = 408
_C = 80
_CPT0 = 108
_CPT1 = 20
_RPT = _NP // 16


def _tbl_body(x_ref, rp_ref, w_ref, xt_ref, rt_ref):
    w = w_ref[0]
    xt_ref[0] = jnp.dot(x_ref[...], w, preferred_element_type=jnp.float32)
    rt_ref[0] = -jnp.dot(rp_ref[...], w, preferred_element_type=jnp.float32)


def _tbl_call(x, rp, w3):
    return pl.pallas_call(
        _tbl_body,
        grid=(3,),
        in_specs=[
            pl.BlockSpec((_NP, _D), lambda k: (0, 0)),
            pl.BlockSpec((_RP, _D), lambda k: (0, 0)),
            pl.BlockSpec((1, _D, _D), lambda k: (k, 0, 0)),
        ],
        out_specs=[
            pl.BlockSpec((1, _NP, _D), lambda k: (k, 0, 0)),
            pl.BlockSpec((1, _RP, _D), lambda k: (k, 0, 0)),
        ],
        out_shape=[
            jax.ShapeDtypeStruct((3, _NP, _D), jnp.float32),
            jax.ShapeDtypeStruct((3, _RP, _D), jnp.float32),
        ],
    )(x, rp, w3)


def _bn_body(p_ref, g_ref, b_ref, rp_ref, rw_ref, x_ref, rn_ref):
    h = p_ref[0] + p_ref[1]
    m = jnp.sum(h, axis=0, keepdims=True) * (1.0 / _N)
    d = h - m
    v = jnp.sum(d * d, axis=0, keepdims=True) * (1.0 / _N) \
        - (float(_NP - _N) / _N) * (m * m)
    xn = jnp.tanh(d / jnp.sqrt(v + 1e-5) * g_ref[...] + b_ref[...])
    row = lax.broadcasted_iota(jnp.int32, (_NP, 1), 0)
    x_ref[...] = jnp.where(row < _N, xn, 0.0)
    rn_ref[...] = jnp.dot(rp_ref[...], rw_ref[...],
                          preferred_element_type=jnp.float32)


def _bn_call(p, g, b, rp, rw):
    return pl.pallas_call(
        _bn_body,
        out_shape=[
            jax.ShapeDtypeStruct((_NP, _D), jnp.float32),
            jax.ShapeDtypeStruct((_RP, _D), jnp.float32),
        ],
    )(p, g, b, rp, rw)


@functools.cache
def _make_sc_scatter():
    mesh = plsc.VectorSubcoreMesh(core_axis_name="c", subcore_axis_name="s")

    @functools.partial(
        pl.kernel,
        mesh=mesh,
        out_type=jax.ShapeDtypeStruct((2 * _NP, _D), jnp.float32),
        scratch_types=[
            pltpu.VMEM((_C,), jnp.int32),
            pltpu.VMEM((_C,), jnp.int32),
            pltpu.VMEM((_C,), jnp.int32),
            pltpu.VMEM((_C,), jnp.int32),
            pltpu.VMEM((_C,), jnp.int32),
            pltpu.VMEM((_C,), jnp.int32),
            pltpu.VMEM((_C, _D), jnp.float32),
            pltpu.VMEM((_C, _D), jnp.float32),
            pltpu.VMEM((_C, _D), jnp.float32),
            pltpu.VMEM((_C, _D), jnp.float32),
            pltpu.VMEM_SHARED((_NP, _D), jnp.float32),
            pltpu.SemaphoreType.DMA,
            pltpu.SemaphoreType.DMA,
            pltpu.SemaphoreType.DMA,
            pltpu.SemaphoreType.DMA,
        ],
    )
    def _sc_scatter(xtbl, nrtbl, gsrc, grel, dste, zer, out,
                    gi0, ri0, di0, gi1, ri1, di1, ra0, rb0, ra1, rb1,
                    acc, si0, si1, sg0, sg1):
        cid = lax.axis_index("c")
        sid = lax.axis_index("s")
        pltpu.sync_copy(zer, acc.at[pl.ds(sid * _RPT, _RPT)])
        ncht = jnp.where(cid == 0, _CPT0, _CPT1)
        cbase = jnp.where(cid == 0, sid * _CPT0, 16 * _CPT0 + sid * _CPT1)
        base = cbase * _C
        isets = ((gi0, ri0, di0, si0), (gi1, ri1, di1, si1))
        rsets = ((ra0, rb0, sg0), (ra1, rb1, sg1))

        def fire_idx(j, s):
            gi, ri, di, si = isets[s]
            o = pl.multiple_of(base + j * _C, 8)
            pltpu.async_copy(gsrc.at[pl.ds(o, _C)], gi, si)
            pltpu.async_copy(grel.at[pl.ds(o, _C)], ri, si)
            pltpu.async_copy(dste.at[pl.ds(o, _C)], di, si)

        def drain_idx(s):
            gi, ri, di, si = isets[s]
            pltpu.make_async_copy(gsrc.at[pl.ds(0, _C)], gi, si).wait()
            pltpu.make_async_copy(grel.at[pl.ds(0, _C)], ri, si).wait()
            pltpu.make_async_copy(dste.at[pl.ds(0, _C)], di, si).wait()

        def fire_rows(s):
            gi, ri, _, _ = isets[s]
            ra, rb, sg = rsets[s]
            pltpu.async_copy(xtbl.at[gi], ra, sg)
            pltpu.async_copy(nrtbl.at[ri], rb, sg)

        def drain_rows(s):
            gi, ri, _, _ = isets[s]
            ra, rb, sg = rsets[s]
            pltpu.make_async_copy(xtbl.at[gi], ra, sg).wait()
            pltpu.make_async_copy(nrtbl.at[ri], rb, sg).wait()

        fire_idx(0, 0)
        fire_idx(1, 1)
        drain_idx(0)
        fire_rows(0)
        plsc.subcore_barrier()

        def body(j2, carry):
            for b in range(2):
                j = j2 * 2 + b
                nb = 1 - b

                @pl.when(j + 1 < ncht)
                def _():
                    drain_idx(nb)
                drain_rows(b)

                @pl.when(j + 1 < ncht)
                def _():
                    fire_rows(nb)
                di = isets[b][2]
                ra, rb, _ = rsets[b]
                pltpu.sync_copy(ra, acc.at[di], add=True)
                pltpu.sync_copy(rb, acc.at[di], add=True)

                @pl.when(j + 2 < ncht)
                def _():
                    fire_idx(j + 2, b)
            return carry

        lax.fori_loop(0, ncht // 2, body, 0)
        plsc.subcore_barrier()
        off = pl.multiple_of(cid * _NP + sid * _RPT, _RPT)
        pltpu.sync_copy(acc.at[pl.ds(sid * _RPT, _RPT)],
                        out.at[pl.ds(off, _RPT)])

    return _sc_scatter


def kernel(ent_ids, edge_index, edge_type, y, entity_embeds, rel_embeds,
           self_rel_embed, weights1, relation_weight1, bn1_gamma, bn1_beta,
           weights2, relation_weight2, bn2_gamma, bn2_beta):
    f32 = jnp.float32
    x0 = jnp.take(entity_embeds, ent_ids, axis=0)
    x0 = jnp.concatenate([x0, jnp.zeros((_NP - _N, _D), f32)], axis=0)
    r0 = jnp.concatenate(
        [rel_embeds, -rel_embeds, self_rel_embed,
         jnp.zeros((_RP - _R2, _D), f32)], axis=0)
    src = edge_index[0]
    dstv = edge_index[1]
    pad = _EP - _E
    gsrc = jnp.concatenate([y * _NP + src, jnp.full((pad,), _N, jnp.int32)])
    grel = jnp.concatenate([y * _RP + edge_type,
                            jnp.full((pad,), _R2, jnp.int32)])
    dste = jnp.concatenate([dstv, jnp.full((pad,), _N + 100, jnp.int32)])
    zer = jnp.zeros((_RPT, _D), f32)
    g1 = bn1_gamma.reshape(1, _D)
    b1 = bn1_beta.reshape(1, _D)
    g2 = bn2_gamma.reshape(1, _D)
    b2 = bn2_beta.reshape(1, _D)

    sc_scatter = _make_sc_scatter()
    xt, rt = _tbl_call(x0, r0, weights1)
    part = sc_scatter(xt.reshape(3 * _NP, _D), rt.reshape(3 * _RP, _D),
                      gsrc, grel, dste, zer)
    x1, r1 = _bn_call(part.reshape(2, _NP, _D), g1, b1, r0, relation_weight1)

    xt2, rt2 = _tbl_call(x1, r1, weights2)
    part2 = sc_scatter(xt2.reshape(3 * _NP, _D), rt2.reshape(3 * _RP, _D),
                       gsrc, grel, dste, zer)
    x2, r2 = _bn_call(part2.reshape(2, _NP, _D), g2, b2, r1, relation_weight2)

    return x2[:_N], r2[:_R2]


# speedup vs baseline: 1.0558x; 1.0319x over previous
import functools

import jax
import jax.numpy as jnp
from jax import lax
from jax.experimental import pallas as pl
from jax.experimental.pallas import tpu as pltpu
from jax.experimental.pallas import tpu_sc as plsc

_N = 10000
_NP = 10240
_E = 160000
_EP = 163840
_D = 128
_R2 = 401
_RP = 408
_C = 80
_CPT0 = 104
_CPT1 = 24
_RPT = _NP // 16


def _tbl_body(x_ref, rp_ref, w_ref, xt_ref, rt_ref):
    w = w_ref[0]
    xt_ref[0] = jnp.dot(x_ref[...], w, preferred_element_type=jnp.float32)
    rt_ref[0] = -jnp.dot(rp_ref[...], w, preferred_element_type=jnp.float32)


def _tbl_call(x, rp, w3):
    return pl.pallas_call(
        _tbl_body,
        grid=(3,),
        in_specs=[
            pl.BlockSpec((_NP, _D), lambda k: (0, 0)),
            pl.BlockSpec((_RP, _D), lambda k: (0, 0)),
            pl.BlockSpec((1, _D, _D), lambda k: (k, 0, 0)),
        ],
        out_specs=[
            pl.BlockSpec((1, _NP, _D), lambda k: (k, 0, 0)),
            pl.BlockSpec((1, _RP, _D), lambda k: (k, 0, 0)),
        ],
        out_shape=[
            jax.ShapeDtypeStruct((3, _NP, _D), jnp.float32),
            jax.ShapeDtypeStruct((3, _RP, _D), jnp.float32),
        ],
    )(x, rp, w3)


def _bn_body(p_ref, g_ref, b_ref, rp_ref, rw_ref, x_ref, rn_ref):
    h = p_ref[0] + p_ref[1]
    m = jnp.sum(h, axis=0, keepdims=True) * (1.0 / _N)
    d = h - m
    v = jnp.sum(d * d, axis=0, keepdims=True) * (1.0 / _N) \
        - (float(_NP - _N) / _N) * (m * m)
    xn = jnp.tanh(d / jnp.sqrt(v + 1e-5) * g_ref[...] + b_ref[...])
    row = lax.broadcasted_iota(jnp.int32, (_NP, 1), 0)
    x_ref[...] = jnp.where(row < _N, xn, 0.0)
    rn_ref[...] = jnp.dot(rp_ref[...], rw_ref[...],
                          preferred_element_type=jnp.float32)


def _bn_call(p, g, b, rp, rw):
    return pl.pallas_call(
        _bn_body,
        out_shape=[
            jax.ShapeDtypeStruct((_NP, _D), jnp.float32),
            jax.ShapeDtypeStruct((_RP, _D), jnp.float32),
        ],
    )(p, g, b, rp, rw)


@functools.cache
def _make_sc_scatter():
    mesh = plsc.VectorSubcoreMesh(core_axis_name="c", subcore_axis_name="s")

    @functools.partial(
        pl.kernel,
        mesh=mesh,
        out_type=jax.ShapeDtypeStruct((2 * _NP, _D), jnp.float32),
        scratch_types=[
            pltpu.VMEM((_C,), jnp.int32),
            pltpu.VMEM((_C,), jnp.int32),
            pltpu.VMEM((_C,), jnp.int32),
            pltpu.VMEM((_C,), jnp.int32),
            pltpu.VMEM((_C,), jnp.int32),
            pltpu.VMEM((_C,), jnp.int32),
            pltpu.VMEM((_C, _D), jnp.float32),
            pltpu.VMEM((_C, _D), jnp.float32),
            pltpu.VMEM((_C, _D), jnp.float32),
            pltpu.VMEM((_C, _D), jnp.float32),
            pltpu.VMEM_SHARED((_NP, _D), jnp.float32),
            pltpu.SemaphoreType.DMA,
            pltpu.SemaphoreType.DMA,
            pltpu.SemaphoreType.DMA,
            pltpu.SemaphoreType.DMA,
        ],
    )
    def _sc_scatter(xtbl, nrtbl, gsrc, grel, dste, zer, out,
                    gi0, ri0, di0, gi1, ri1, di1, ra0, rb0, ra1, rb1,
                    acc, si0, si1, sg0, sg1):
        cid = lax.axis_index("c")
        sid = lax.axis_index("s")
        pltpu.sync_copy(zer, acc.at[pl.ds(sid * _RPT, _RPT)])
        ncht = jnp.where(cid == 0, _CPT0, _CPT1)
        cbase = jnp.where(cid == 0, sid * _CPT0, 16 * _CPT0 + sid * _CPT1)
        base = cbase * _C
        isets = ((gi0, ri0, di0, si0), (gi1, ri1, di1, si1))
        rsets = ((ra0, rb0, sg0), (ra1, rb1, sg1))

        def fire_idx(j, s):
            gi, ri, di, si = isets[s]
            o = pl.multiple_of(base + j * _C, 8)
            pltpu.async_copy(gsrc.at[pl.ds(o, _C)], gi, si)
            pltpu.async_copy(grel.at[pl.ds(o, _C)], ri, si)
            pltpu.async_copy(dste.at[pl.ds(o, _C)], di, si)

        def drain_idx(s):
            gi, ri, di, si = isets[s]
            pltpu.make_async_copy(gsrc.at[pl.ds(0, _C)], gi, si).wait()
            pltpu.make_async_copy(grel.at[pl.ds(0, _C)], ri, si).wait()
            pltpu.make_async_copy(dste.at[pl.ds(0, _C)], di, si).wait()

        def fire_rows(s):
            gi, ri, _, _ = isets[s]
            ra, rb, sg = rsets[s]
            pltpu.async_copy(xtbl.at[gi], ra, sg)
            pltpu.async_copy(nrtbl.at[ri], rb, sg)

        def drain_rows(s):
            gi, ri, _, _ = isets[s]
            ra, rb, sg = rsets[s]
            pltpu.make_async_copy(xtbl.at[gi], ra, sg).wait()
            pltpu.make_async_copy(nrtbl.at[ri], rb, sg).wait()

        fire_idx(0, 0)
        fire_idx(1, 1)
        drain_idx(0)
        fire_rows(0)
        plsc.subcore_barrier()

        def body(j2, carry):
            for b in range(2):
                j = j2 * 2 + b
                nb = 1 - b

                @pl.when(j + 1 < ncht)
                def _():
                    drain_idx(nb)
                drain_rows(b)

                @pl.when(j + 1 < ncht)
                def _():
                    fire_rows(nb)
                di = isets[b][2]
                ra, rb, _ = rsets[b]
                pltpu.sync_copy(ra, acc.at[di], add=True)
                pltpu.sync_copy(rb, acc.at[di], add=True)

                @pl.when(j + 2 < ncht)
                def _():
                    fire_idx(j + 2, b)
            return carry

        lax.fori_loop(0, ncht // 2, body, 0)
        plsc.subcore_barrier()
        off = pl.multiple_of(cid * _NP + sid * _RPT, _RPT)
        pltpu.sync_copy(acc.at[pl.ds(sid * _RPT, _RPT)],
                        out.at[pl.ds(off, _RPT)])

    return _sc_scatter


def kernel(ent_ids, edge_index, edge_type, y, entity_embeds, rel_embeds,
           self_rel_embed, weights1, relation_weight1, bn1_gamma, bn1_beta,
           weights2, relation_weight2, bn2_gamma, bn2_beta):
    f32 = jnp.float32
    x0 = jnp.take(entity_embeds, ent_ids, axis=0)
    x0 = jnp.concatenate([x0, jnp.zeros((_NP - _N, _D), f32)], axis=0)
    r0 = jnp.concatenate(
        [rel_embeds, -rel_embeds, self_rel_embed,
         jnp.zeros((_RP - _R2, _D), f32)], axis=0)
    src = edge_index[0]
    dstv = edge_index[1]
    pad = _EP - _E
    gsrc = jnp.concatenate([y * _NP + src, jnp.full((pad,), _N, jnp.int32)])
    grel = jnp.concatenate([y * _RP + edge_type,
                            jnp.full((pad,), _R2, jnp.int32)])
    dste = jnp.concatenate([dstv, jnp.full((pad,), _N + 100, jnp.int32)])
    zer = jnp.zeros((_RPT, _D), f32)
    g1 = bn1_gamma.reshape(1, _D)
    b1 = bn1_beta.reshape(1, _D)
    g2 = bn2_gamma.reshape(1, _D)
    b2 = bn2_beta.reshape(1, _D)

    sc_scatter = _make_sc_scatter()
    xt, rt = _tbl_call(x0, r0, weights1)
    part = sc_scatter(xt.reshape(3 * _NP, _D), rt.reshape(3 * _RP, _D),
                      gsrc, grel, dste, zer)
    x1, r1 = _bn_call(part.reshape(2, _NP, _D), g1, b1, r0, relation_weight1)

    xt2, rt2 = _tbl_call(x1, r1, weights2)
    part2 = sc_scatter(xt2.reshape(3 * _NP, _D), rt2.reshape(3 * _RP, _D),
                       gsrc, grel, dste, zer)
    x2, r2 = _bn_call(part2.reshape(2, _NP, _D), g2, b2, r1, relation_weight2)

    return x2[:_N], r2[:_R2]
